# baseline (speedup 1.0000x reference)
import functools

import jax
import jax.numpy as jnp
from jax import lax
from jax.experimental import pallas as pl
from jax.experimental.pallas import tpu as pltpu
from jax.experimental.pallas import tpu_sc as plsc

D = 128
N = 100000
NC, NS = 2, 16
NW = NC * NS
CH = 128
NFULL = N // CH
TAIL = N - NFULL * CH
MAXK = (NFULL + NW - 1) // NW


def _embed_body(table_hbm, idx_hbm, out_hbm, idx_v, rows_v, sem):
    wid = lax.axis_index("s") * NC + lax.axis_index("c")

    def step(k, carry):
        c = wid + k * NW

        @pl.when(c < NFULL)
        def _():
            base = pl.multiple_of(c * CH, CH)
            pltpu.sync_copy(idx_hbm.at[pl.ds(base, CH)], idx_v)
            pltpu.async_copy(table_hbm.at[idx_v], rows_v, sem).wait()
            pltpu.sync_copy(rows_v, out_hbm.at[pl.ds(base, CH)])

        return carry

    lax.fori_loop(0, MAXK, step, 0)

    @pl.when(wid == NW - 1)
    def _tail():
        base = NFULL * CH
        pltpu.sync_copy(idx_hbm.at[pl.ds(base, TAIL)], idx_v.at[pl.ds(0, TAIL)])
        pltpu.async_copy(
            table_hbm.at[idx_v.at[pl.ds(0, TAIL)]],
            rows_v.at[pl.ds(0, TAIL)],
            sem,
        ).wait()
        pltpu.sync_copy(rows_v.at[pl.ds(0, TAIL)], out_hbm.at[pl.ds(base, TAIL)])


_embed = functools.partial(
    pl.kernel,
    out_type=jax.ShapeDtypeStruct((N, D), jnp.float32),
    mesh=plsc.VectorSubcoreMesh(
        core_axis_name="c", subcore_axis_name="s", num_cores=NC, num_subcores=NS
    ),
    scratch_types=[
        pltpu.VMEM((CH,), jnp.int32),
        pltpu.VMEM((CH, D), jnp.float32),
        pltpu.SemaphoreType.DMA,
    ],
)(_embed_body)


def kernel(atom_types, W):
    idx = atom_types.astype(jnp.int32)
    table = W.T
    return _embed(table, idx)
